# Initial kernel scaffold; baseline (speedup 1.0000x reference)
import jax
import jax.numpy as jnp
from jax.experimental import pallas as pl


def kernel(edge_index, x, pos, edge_attr, params):
    raise NotImplementedError("write your pallas kernel here")


# speedup vs baseline: 2.1993x; 2.1993x over previous
import functools

import jax
import jax.numpy as jnp
from jax import lax
from jax.experimental import pallas as pl
from jax.experimental.pallas import tpu as pltpu
from jax.experimental.pallas import tpu_sc as plsc

F32 = jnp.float32

_NC = 2
_NS = 16
_NW = _NC * _NS

_CHUNK = 80

_MESH = dict(core_axis_name="c", subcore_axis_name="s")
_SC_PARAMS = pltpu.CompilerParams(use_tc_tiling_on_sc=False)


def _silu(v):
    return v * jax.nn.sigmoid(v)


def _tc_embed(x, We, be, W1a, W1b, b1):
    N, IC = x.shape
    H = We.shape[1]
    BN = 1000
    grid = (N // BN,)

    def body(x_r, We_r, be_r, W1a_r, W1b_r, b1_r, h_r, A_r, B_r):
        h = jnp.dot(x_r[...], We_r[...], preferred_element_type=F32) + be_r[...]
        h_r[...] = h
        A_r[...] = jnp.dot(h, W1a_r[...], preferred_element_type=F32) + b1_r[...]
        B_r[...] = jnp.dot(h, W1b_r[...], preferred_element_type=F32)

    full = lambda a, b: pl.BlockSpec((a, b), lambda i: (0, 0))
    return pl.pallas_call(
        body,
        grid=grid,
        in_specs=[
            pl.BlockSpec((BN, IC), lambda i: (i, 0)),
            full(IC, H), full(1, H), full(H, H), full(H, H), full(1, H),
        ],
        out_specs=[
            pl.BlockSpec((BN, H), lambda i: (i, 0)),
            pl.BlockSpec((BN, H), lambda i: (i, 0)),
            pl.BlockSpec((BN, H), lambda i: (i, 0)),
        ],
        out_shape=[
            jax.ShapeDtypeStruct((N, H), F32),
            jax.ShapeDtypeStruct((N, H), F32),
            jax.ShapeDtypeStruct((N, H), F32),
        ],
    )(x, We, be, W1a, W1b, b1)


def _tc_edge(preA, preB, Pd, Ps, ea, w1c, W1d, W2, b2, Wp1, bp1, wp2, bp2):
    E, H = preA.shape
    ED = ea.shape[1]
    BE = 1000
    grid = (E // BE,)

    def body(pa_r, pb_r, pd_r, ps_r, ea_r, w1c_r, W1d_r, W2_r, b2_r,
             Wp1_r, bp1_r, wp2_r, bp2_r, m_r, relw_r):
        rel = pd_r[...] - ps_r[...]
        d2 = jnp.sum(rel * rel, axis=1, keepdims=True)
        z1 = (pa_r[...] + pb_r[...] + d2 * w1c_r[...]
              + jnp.dot(ea_r[...], W1d_r[...], preferred_element_type=F32))
        t = _silu(z1)
        m = _silu(jnp.dot(t, W2_r[...], preferred_element_type=F32) + b2_r[...])
        u = _silu(jnp.dot(m, Wp1_r[...], preferred_element_type=F32) + bp1_r[...])
        w = jnp.sum(u * wp2_r[...], axis=1, keepdims=True) + bp2_r[...]
        lane = lax.broadcasted_iota(jnp.int32, (1, 16), 1)
        cmark = (lane == 3).astype(F32)
        m_r[...] = m
        relw_r[...] = rel * w + cmark

    full = lambda a, b: pl.BlockSpec((a, b), lambda i: (0, 0))
    return pl.pallas_call(
        body,
        grid=grid,
        in_specs=[
            pl.BlockSpec((BE, H), lambda i: (i, 0)),
            pl.BlockSpec((BE, H), lambda i: (i, 0)),
            pl.BlockSpec((BE, 16), lambda i: (i, 0)),
            pl.BlockSpec((BE, 16), lambda i: (i, 0)),
            pl.BlockSpec((BE, ED), lambda i: (i, 0)),
            full(1, H), full(ED, H), full(H, H), full(1, H),
            full(H, H), full(1, H), full(1, H), full(1, 1),
        ],
        out_specs=[
            pl.BlockSpec((BE, H), lambda i: (i, 0)),
            pl.BlockSpec((BE, 16), lambda i: (i, 0)),
        ],
        out_shape=[
            jax.ShapeDtypeStruct((E, H), F32),
            jax.ShapeDtypeStruct((E, 16), F32),
        ],
    )(preA, preB, Pd, Ps, ea, w1c, W1d, W2, b2, Wp1, bp1, wp2, bp2)


def _tc_node(h, pM0, pM1, pR0, pR1, posp, Wn1a, Wn1b, bn1, Wn2, bn2,
             nW1a, nW1b, nb1):
    N, H = h.shape
    BN = 1000
    grid = (N // BN,)
    has_next = nW1a is not None

    def body(*refs):
        (h_r, pM0_r, pM1_r, pR0_r, pR1_r, posp_r, Wn1a_r, Wn1b_r, bn1_r,
         Wn2_r, bn2_r) = refs[:11]
        k = 11
        if has_next:
            nW1a_r, nW1b_r, nb1_r = refs[k:k + 3]
            k += 3
        out = refs[k:]
        accM = pM0_r[...] + pM1_r[...]
        accR = pR0_r[...] + pR1_r[...]
        lane = lax.broadcasted_iota(jnp.int32, (1, 16), 1)
        cnt = jnp.sum(jnp.where(lane == 3, accR, 0.0), axis=1, keepdims=True)
        colmask = (lane < 3).astype(F32)
        upd = accR * colmask / jnp.maximum(cnt, 1.0)
        t = _silu(jnp.dot(h_r[...], Wn1a_r[...], preferred_element_type=F32)
                  + jnp.dot(accM, Wn1b_r[...], preferred_element_type=F32)
                  + bn1_r[...])
        hn = jnp.dot(t, Wn2_r[...], preferred_element_type=F32) + bn2_r[...]
        out[0][...] = hn
        out[1][...] = posp_r[...] + upd
        if has_next:
            out[2][...] = jnp.dot(hn, nW1a_r[...], preferred_element_type=F32) + nb1_r[...]
            out[3][...] = jnp.dot(hn, nW1b_r[...], preferred_element_type=F32)

    full = lambda a, b: pl.BlockSpec((a, b), lambda i: (0, 0))
    row = lambda w: pl.BlockSpec((BN, w), lambda i: (i, 0))
    in_specs = [row(H), row(H), row(H), row(16), row(16), row(16),
                full(H, H), full(H, H), full(1, H), full(H, H), full(1, H)]
    ins = [h, pM0, pM1, pR0, pR1, posp, Wn1a, Wn1b, bn1, Wn2, bn2]
    out_specs = [row(H), row(16)]
    out_shape = [jax.ShapeDtypeStruct((N, H), F32),
                 jax.ShapeDtypeStruct((N, 16), F32)]
    if has_next:
        in_specs += [full(H, H), full(H, H), full(1, H)]
        ins += [nW1a, nW1b, nb1]
        out_specs += [row(H), row(H)]
        out_shape += [jax.ShapeDtypeStruct((N, H), F32),
                      jax.ShapeDtypeStruct((N, H), F32)]
    return pl.pallas_call(
        body, grid=grid, in_specs=in_specs, out_specs=out_specs,
        out_shape=out_shape,
    )(*ins)


def _sc_gather(A, B, P, src, dst):
    N, H = A.shape
    E = src.shape[0]
    epw = E // _NW
    nch = epw // _CHUNK
    C = _CHUNK

    mesh = plsc.VectorSubcoreMesh(**_MESH)

    @functools.partial(
        pl.kernel,
        out_type=(
            jax.ShapeDtypeStruct((E, H), F32),
            jax.ShapeDtypeStruct((E, H), F32),
            jax.ShapeDtypeStruct((E, 16), F32),
            jax.ShapeDtypeStruct((E, 16), F32),
        ),
        mesh=mesh,
        compiler_params=_SC_PARAMS,
        scratch_types=[
            pltpu.VMEM((C,), jnp.int32),
            pltpu.VMEM((C,), jnp.int32),
            pltpu.VMEM((C, H), F32),
            pltpu.VMEM((C, H), F32),
            pltpu.VMEM((C, 16), F32),
            pltpu.VMEM((C, 16), F32),
            pltpu.SemaphoreType.DMA,
        ],
    )
    def k(A_h, B_h, P_h, src_h, dst_h, oA, oB, oPd, oPs,
          idxd, idxs, bA, bB, bPd, bPs, sem):
        wid = lax.axis_index("s") * _NC + lax.axis_index("c")

        def chunk(ci, carry):
            base = wid * epw + ci * C
            pltpu.sync_copy(dst_h.at[pl.ds(base, C)], idxd)
            pltpu.sync_copy(src_h.at[pl.ds(base, C)], idxs)
            c1 = pltpu.async_copy(A_h.at[idxd], bA, sem)
            c2 = pltpu.async_copy(B_h.at[idxs], bB, sem)
            c3 = pltpu.async_copy(P_h.at[idxd], bPd, sem)
            c4 = pltpu.async_copy(P_h.at[idxs], bPs, sem)
            c1.wait()
            c2.wait()
            c3.wait()
            c4.wait()
            pltpu.sync_copy(bA, oA.at[pl.ds(base, C)])
            pltpu.sync_copy(bB, oB.at[pl.ds(base, C)])
            pltpu.sync_copy(bPd, oPd.at[pl.ds(base, C)])
            pltpu.sync_copy(bPs, oPs.at[pl.ds(base, C)])
            return carry

        lax.fori_loop(0, nch, chunk, 0)

    return k(A, B, P, src, dst)


def _sc_scatter(m, relw, dst, zM, zR):
    E, H = m.shape
    N = zM.shape[0]
    epw = E // _NW
    nch = epw // _CHUNK
    C = _CHUNK
    rpw = N // _NS

    mesh = plsc.VectorSubcoreMesh(**_MESH)

    @functools.partial(
        pl.kernel,
        out_type=(
            jax.ShapeDtypeStruct((_NC * N, H), F32),
            jax.ShapeDtypeStruct((_NC * N, 16), F32),
        ),
        mesh=mesh,
        compiler_params=_SC_PARAMS,
        scratch_types=[
            pltpu.VMEM((C,), jnp.int32),
            pltpu.VMEM((C, H), F32),
            pltpu.VMEM((C, 16), F32),
            pltpu.VMEM_SHARED((N, H), F32),
            pltpu.VMEM_SHARED((N, 16), F32),
            pltpu.SemaphoreType.DMA,
        ],
    )
    def k(m_h, relw_h, dst_h, zM_h, zR_h, oM, oR,
          idx, bM, bR, accM, accR, sem):
        cid = lax.axis_index("c")
        sid = lax.axis_index("s")
        wid = sid * _NC + cid
        r0 = sid * rpw
        pltpu.sync_copy(zM_h.at[pl.ds(r0, rpw)], accM.at[pl.ds(r0, rpw)])
        pltpu.sync_copy(zR_h.at[pl.ds(r0, rpw)], accR.at[pl.ds(r0, rpw)])
        plsc.subcore_barrier()

        def chunk(ci, carry):
            base = wid * epw + ci * C
            pltpu.sync_copy(dst_h.at[pl.ds(base, C)], idx)
            pltpu.sync_copy(m_h.at[pl.ds(base, C)], bM)
            pltpu.sync_copy(relw_h.at[pl.ds(base, C)], bR)
            pltpu.sync_copy(bM, accM.at[idx], add=True)
            pltpu.sync_copy(bR, accR.at[idx], add=True)
            return carry

        lax.fori_loop(0, nch, chunk, 0)
        plsc.subcore_barrier()
        pltpu.sync_copy(accM.at[pl.ds(r0, rpw)], oM.at[pl.ds(cid * N + r0, rpw)])
        pltpu.sync_copy(accR.at[pl.ds(r0, rpw)], oR.at[pl.ds(cid * N + r0, rpw)])

    return k(m, relw, dst, zM, zR)


def kernel(edge_index, x, pos, edge_attr, params):
    src = edge_index[0]
    dst = edge_index[1]
    N = x.shape[0]
    H = params["emb"]["W"][0].shape[1]

    posp = jnp.pad(pos, ((0, 0), (0, 16 - pos.shape[1])))
    zM = jnp.zeros((N, H), F32)
    zR = jnp.zeros((N, 16), F32)

    layers = params["layers"]
    emb = params["emb"]

    def edge_w(lp):
        W1 = lp["edge"]["W"][0]
        return (W1[:H], W1[H:2 * H], W1[2 * H:2 * H + 1],
                W1[2 * H + 1:], lp["edge"]["b"][0].reshape(1, H))

    W1a0, W1b0, _, _, b10 = edge_w(layers[0])
    h, A, B = _tc_embed(x, emb["W"][0], emb["b"][0].reshape(1, H),
                        W1a0, W1b0, b10)

    for li, lp in enumerate(layers):
        _, _, w1c, W1d, _ = edge_w(lp)
        preA, preB, Pd, Ps = _sc_gather(A, B, posp, src, dst)
        m, relw = _tc_edge(
            preA, preB, Pd, Ps, edge_attr,
            w1c, W1d,
            lp["edge"]["W"][1], lp["edge"]["b"][1].reshape(1, H),
            lp["pos"]["W"][0], lp["pos"]["b"][0].reshape(1, H),
            lp["pos"]["W"][1].reshape(1, H), lp["pos"]["b"][1].reshape(1, 1),
        )
        pM, pR = _sc_scatter(m, relw, dst, zM, zR)
        Wn1 = lp["node"]["W"][0]
        is_last = li == len(layers) - 1
        if not is_last:
            nW1a, nW1b, _, _, nb1 = edge_w(layers[li + 1])
        else:
            nW1a = nW1b = nb1 = None
        outs = _tc_node(
            h, pM[:N], pM[N:], pR[:N], pR[N:], posp,
            Wn1[:H], Wn1[H:], lp["node"]["b"][0].reshape(1, H),
            lp["node"]["W"][1], lp["node"]["b"][1].reshape(1, H),
            nW1a, nW1b, nb1,
        )
        if not is_last:
            h, posp, A, B = outs
        else:
            h, posp = outs

    return h, posp[:, :3]


# speedup vs baseline: 2.4106x; 1.0961x over previous
import functools

import jax
import jax.numpy as jnp
from jax import lax
from jax.experimental import pallas as pl
from jax.experimental.pallas import tpu as pltpu
from jax.experimental.pallas import tpu_sc as plsc

F32 = jnp.float32

_NC = 2
_NS = 16
_NW = _NC * _NS

_CHUNK = 80

_MESH = dict(core_axis_name="c", subcore_axis_name="s")
_SC_PARAMS = pltpu.CompilerParams(use_tc_tiling_on_sc=False)


def _silu(v):
    return v * jax.nn.sigmoid(v)


def _tc_embed(x, We, be, W1a, W1b, b1):
    N, IC = x.shape
    H = We.shape[1]
    BN = 2000
    grid = (N // BN,)

    def body(x_r, We_r, be_r, W1a_r, W1b_r, b1_r, h_r, A_r, B_r):
        h = jnp.dot(x_r[...], We_r[...], preferred_element_type=F32) + be_r[...]
        h_r[...] = h
        A_r[...] = jnp.dot(h, W1a_r[...], preferred_element_type=F32) + b1_r[...]
        B_r[...] = jnp.dot(h, W1b_r[...], preferred_element_type=F32)

    full = lambda a, b: pl.BlockSpec((a, b), lambda i: (0, 0))
    return pl.pallas_call(
        body,
        grid=grid,
        in_specs=[
            pl.BlockSpec((BN, IC), lambda i: (i, 0)),
            full(IC, H), full(1, H), full(H, H), full(H, H), full(1, H),
        ],
        out_specs=[
            pl.BlockSpec((BN, H), lambda i: (i, 0)),
            pl.BlockSpec((BN, H), lambda i: (i, 0)),
            pl.BlockSpec((BN, H), lambda i: (i, 0)),
        ],
        out_shape=[
            jax.ShapeDtypeStruct((N, H), F32),
            jax.ShapeDtypeStruct((N, H), F32),
            jax.ShapeDtypeStruct((N, H), F32),
        ],
    )(x, We, be, W1a, W1b, b1)


def _tc_edge(preA, preB, Pd, Ps, ea, w1c, W1d, W2, b2, Wp1, bp1, wp2, bp2):
    E, H = preA.shape
    ED = ea.shape[1]
    BE = 2000
    grid = (E // BE,)

    def body(pa_r, pb_r, pd_r, ps_r, ea_r, w1c_r, W1d_r, W2_r, b2_r,
             Wp1_r, bp1_r, wp2_r, bp2_r, m_r, relw_r):
        rel = pd_r[...] - ps_r[...]
        d2 = jnp.sum(rel * rel, axis=1, keepdims=True)
        z1 = (pa_r[...] + pb_r[...] + d2 * w1c_r[...]
              + jnp.dot(ea_r[...], W1d_r[...], preferred_element_type=F32))
        t = _silu(z1)
        m = _silu(jnp.dot(t, W2_r[...], preferred_element_type=F32) + b2_r[...])
        u = _silu(jnp.dot(m, Wp1_r[...], preferred_element_type=F32) + bp1_r[...])
        w = jnp.sum(u * wp2_r[...], axis=1, keepdims=True) + bp2_r[...]
        lane = lax.broadcasted_iota(jnp.int32, (1, 16), 1)
        cmark = (lane == 3).astype(F32)
        m_r[...] = m
        relw_r[...] = rel * w + cmark

    full = lambda a, b: pl.BlockSpec((a, b), lambda i: (0, 0))
    return pl.pallas_call(
        body,
        grid=grid,
        in_specs=[
            pl.BlockSpec((BE, H), lambda i: (i, 0)),
            pl.BlockSpec((BE, H), lambda i: (i, 0)),
            pl.BlockSpec((BE, 16), lambda i: (i, 0)),
            pl.BlockSpec((BE, 16), lambda i: (i, 0)),
            pl.BlockSpec((BE, ED), lambda i: (i, 0)),
            full(1, H), full(ED, H), full(H, H), full(1, H),
            full(H, H), full(1, H), full(1, H), full(1, 1),
        ],
        out_specs=[
            pl.BlockSpec((BE, H), lambda i: (i, 0)),
            pl.BlockSpec((BE, 16), lambda i: (i, 0)),
        ],
        out_shape=[
            jax.ShapeDtypeStruct((E, H), F32),
            jax.ShapeDtypeStruct((E, 16), F32),
        ],
    )(preA, preB, Pd, Ps, ea, w1c, W1d, W2, b2, Wp1, bp1, wp2, bp2)


def _tc_node(h, pM0, pM1, pR0, pR1, posp, Wn1a, Wn1b, bn1, Wn2, bn2,
             nW1a, nW1b, nb1):
    N, H = h.shape
    BN = 2000
    grid = (N // BN,)
    has_next = nW1a is not None

    def body(*refs):
        (h_r, pM0_r, pM1_r, pR0_r, pR1_r, posp_r, Wn1a_r, Wn1b_r, bn1_r,
         Wn2_r, bn2_r) = refs[:11]
        k = 11
        if has_next:
            nW1a_r, nW1b_r, nb1_r = refs[k:k + 3]
            k += 3
        out = refs[k:]
        accM = pM0_r[...] + pM1_r[...]
        accR = pR0_r[...] + pR1_r[...]
        lane = lax.broadcasted_iota(jnp.int32, (1, 16), 1)
        cnt = jnp.sum(jnp.where(lane == 3, accR, 0.0), axis=1, keepdims=True)
        colmask = (lane < 3).astype(F32)
        upd = accR * colmask / jnp.maximum(cnt, 1.0)
        t = _silu(jnp.dot(h_r[...], Wn1a_r[...], preferred_element_type=F32)
                  + jnp.dot(accM, Wn1b_r[...], preferred_element_type=F32)
                  + bn1_r[...])
        hn = jnp.dot(t, Wn2_r[...], preferred_element_type=F32) + bn2_r[...]
        out[0][...] = hn
        out[1][...] = posp_r[...] + upd
        if has_next:
            out[2][...] = jnp.dot(hn, nW1a_r[...], preferred_element_type=F32) + nb1_r[...]
            out[3][...] = jnp.dot(hn, nW1b_r[...], preferred_element_type=F32)

    full = lambda a, b: pl.BlockSpec((a, b), lambda i: (0, 0))
    row = lambda w: pl.BlockSpec((BN, w), lambda i: (i, 0))
    in_specs = [row(H), row(H), row(H), row(16), row(16), row(16),
                full(H, H), full(H, H), full(1, H), full(H, H), full(1, H)]
    ins = [h, pM0, pM1, pR0, pR1, posp, Wn1a, Wn1b, bn1, Wn2, bn2]
    out_specs = [row(H), row(16)]
    out_shape = [jax.ShapeDtypeStruct((N, H), F32),
                 jax.ShapeDtypeStruct((N, 16), F32)]
    if has_next:
        in_specs += [full(H, H), full(H, H), full(1, H)]
        ins += [nW1a, nW1b, nb1]
        out_specs += [row(H), row(H)]
        out_shape += [jax.ShapeDtypeStruct((N, H), F32),
                      jax.ShapeDtypeStruct((N, H), F32)]
    return pl.pallas_call(
        body, grid=grid, in_specs=in_specs, out_specs=out_specs,
        out_shape=out_shape,
    )(*ins)


def _sc_gather(A, B, P, src, dst):
    N, H = A.shape
    E = src.shape[0]
    epw = E // _NW
    nch = epw // _CHUNK
    C = _CHUNK

    mesh = plsc.VectorSubcoreMesh(**_MESH)

    @functools.partial(
        pl.kernel,
        out_type=(
            jax.ShapeDtypeStruct((E, H), F32),
            jax.ShapeDtypeStruct((E, H), F32),
            jax.ShapeDtypeStruct((E, 16), F32),
            jax.ShapeDtypeStruct((E, 16), F32),
        ),
        mesh=mesh,
        compiler_params=_SC_PARAMS,
        scratch_types=[
            pltpu.VMEM((C,), jnp.int32),
            pltpu.VMEM((C,), jnp.int32),
            pltpu.VMEM((C, H), F32),
            pltpu.VMEM((C, H), F32),
            pltpu.VMEM((C, 16), F32),
            pltpu.VMEM((C, 16), F32),
            pltpu.SemaphoreType.DMA,
        ],
    )
    def k(A_h, B_h, P_h, src_h, dst_h, oA, oB, oPd, oPs,
          idxd, idxs, bA, bB, bPd, bPs, sem):
        wid = lax.axis_index("s") * _NC + lax.axis_index("c")

        def chunk(ci, carry):
            base = wid * epw + ci * C
            pltpu.sync_copy(dst_h.at[pl.ds(base, C)], idxd)
            pltpu.sync_copy(src_h.at[pl.ds(base, C)], idxs)
            c1 = pltpu.async_copy(A_h.at[idxd], bA, sem)
            c2 = pltpu.async_copy(B_h.at[idxs], bB, sem)
            c3 = pltpu.async_copy(P_h.at[idxd], bPd, sem)
            c4 = pltpu.async_copy(P_h.at[idxs], bPs, sem)
            c1.wait()
            c2.wait()
            c3.wait()
            c4.wait()
            pltpu.sync_copy(bA, oA.at[pl.ds(base, C)])
            pltpu.sync_copy(bB, oB.at[pl.ds(base, C)])
            pltpu.sync_copy(bPd, oPd.at[pl.ds(base, C)])
            pltpu.sync_copy(bPs, oPs.at[pl.ds(base, C)])
            return carry

        lax.fori_loop(0, nch, chunk, 0)

    return k(A, B, P, src, dst)


def _sc_scatter(m, relw, dst, zM, zR):
    E, H = m.shape
    N = zM.shape[0]
    epw = E // _NW
    nch = epw // _CHUNK
    C = _CHUNK
    rpw = N // _NS

    mesh = plsc.VectorSubcoreMesh(**_MESH)

    @functools.partial(
        pl.kernel,
        out_type=(
            jax.ShapeDtypeStruct((_NC * N, H), F32),
            jax.ShapeDtypeStruct((_NC * N, 16), F32),
        ),
        mesh=mesh,
        compiler_params=_SC_PARAMS,
        scratch_types=[
            pltpu.VMEM((C,), jnp.int32),
            pltpu.VMEM((C, H), F32),
            pltpu.VMEM((C, 16), F32),
            pltpu.VMEM_SHARED((N, H), F32),
            pltpu.VMEM_SHARED((N, 16), F32),
            pltpu.SemaphoreType.DMA,
        ],
    )
    def k(m_h, relw_h, dst_h, zM_h, zR_h, oM, oR,
          idx, bM, bR, accM, accR, sem):
        cid = lax.axis_index("c")
        sid = lax.axis_index("s")
        wid = sid * _NC + cid
        r0 = sid * rpw
        pltpu.sync_copy(zM_h.at[pl.ds(r0, rpw)], accM.at[pl.ds(r0, rpw)])
        pltpu.sync_copy(zR_h.at[pl.ds(r0, rpw)], accR.at[pl.ds(r0, rpw)])
        plsc.subcore_barrier()

        def chunk(ci, carry):
            base = wid * epw + ci * C
            pltpu.sync_copy(dst_h.at[pl.ds(base, C)], idx)
            pltpu.sync_copy(m_h.at[pl.ds(base, C)], bM)
            pltpu.sync_copy(relw_h.at[pl.ds(base, C)], bR)
            pltpu.sync_copy(bM, accM.at[idx], add=True)
            pltpu.sync_copy(bR, accR.at[idx], add=True)
            return carry

        lax.fori_loop(0, nch, chunk, 0)
        plsc.subcore_barrier()
        pltpu.sync_copy(accM.at[pl.ds(r0, rpw)], oM.at[pl.ds(cid * N + r0, rpw)])
        pltpu.sync_copy(accR.at[pl.ds(r0, rpw)], oR.at[pl.ds(cid * N + r0, rpw)])

    return k(m, relw, dst, zM, zR)


def kernel(edge_index, x, pos, edge_attr, params):
    src = edge_index[0]
    dst = edge_index[1]
    N = x.shape[0]
    H = params["emb"]["W"][0].shape[1]

    posp = jnp.pad(pos, ((0, 0), (0, 16 - pos.shape[1])))
    zM = jnp.zeros((N, H), F32)
    zR = jnp.zeros((N, 16), F32)

    layers = params["layers"]
    emb = params["emb"]

    def edge_w(lp):
        W1 = lp["edge"]["W"][0]
        return (W1[:H], W1[H:2 * H], W1[2 * H:2 * H + 1],
                W1[2 * H + 1:], lp["edge"]["b"][0].reshape(1, H))

    W1a0, W1b0, _, _, b10 = edge_w(layers[0])
    h, A, B = _tc_embed(x, emb["W"][0], emb["b"][0].reshape(1, H),
                        W1a0, W1b0, b10)

    for li, lp in enumerate(layers):
        _, _, w1c, W1d, _ = edge_w(lp)
        preA, preB, Pd, Ps = _sc_gather(A, B, posp, src, dst)
        m, relw = _tc_edge(
            preA, preB, Pd, Ps, edge_attr,
            w1c, W1d,
            lp["edge"]["W"][1], lp["edge"]["b"][1].reshape(1, H),
            lp["pos"]["W"][0], lp["pos"]["b"][0].reshape(1, H),
            lp["pos"]["W"][1].reshape(1, H), lp["pos"]["b"][1].reshape(1, 1),
        )
        pM, pR = _sc_scatter(m, relw, dst, zM, zR)
        Wn1 = lp["node"]["W"][0]
        is_last = li == len(layers) - 1
        if not is_last:
            nW1a, nW1b, _, _, nb1 = edge_w(layers[li + 1])
        else:
            nW1a = nW1b = nb1 = None
        outs = _tc_node(
            h, pM[:N], pM[N:], pR[:N], pR[N:], posp,
            Wn1[:H], Wn1[H:], lp["node"]["b"][0].reshape(1, H),
            lp["node"]["W"][1], lp["node"]["b"][1].reshape(1, H),
            nW1a, nW1b, nb1,
        )
        if not is_last:
            h, posp, A, B = outs
        else:
            h, posp = outs

    return h, posp[:, :3]


# speedup vs baseline: 3.0779x; 1.2768x over previous
import functools

import jax
import jax.numpy as jnp
from jax import lax
from jax.experimental import pallas as pl
from jax.experimental.pallas import tpu as pltpu
from jax.experimental.pallas import tpu_sc as plsc

F32 = jnp.float32

_NC = 2
_NS = 16
_NW = _NC * _NS

_CHUNK = 80

_MESH = dict(core_axis_name="c", subcore_axis_name="s")
_SC_PARAMS = pltpu.CompilerParams(use_tc_tiling_on_sc=False)


def _silu(v):
    return v * jax.nn.sigmoid(v)


def _tc_embed(x, We, be, W1a, W1b, b1):
    N, IC = x.shape
    H = We.shape[1]
    BN = 2000
    grid = (N // BN,)

    def body(x_r, We_r, be_r, W1a_r, W1b_r, b1_r, h_r, A_r, B_r):
        h = jnp.dot(x_r[...], We_r[...], preferred_element_type=F32) + be_r[...]
        h_r[...] = h
        A_r[...] = jnp.dot(h, W1a_r[...], preferred_element_type=F32) + b1_r[...]
        B_r[...] = jnp.dot(h, W1b_r[...], preferred_element_type=F32)

    full = lambda a, b: pl.BlockSpec((a, b), lambda i: (0, 0))
    return pl.pallas_call(
        body,
        grid=grid,
        in_specs=[
            pl.BlockSpec((BN, IC), lambda i: (i, 0)),
            full(IC, H), full(1, H), full(H, H), full(H, H), full(1, H),
        ],
        out_specs=[
            pl.BlockSpec((BN, H), lambda i: (i, 0)),
            pl.BlockSpec((BN, H), lambda i: (i, 0)),
            pl.BlockSpec((BN, H), lambda i: (i, 0)),
        ],
        out_shape=[
            jax.ShapeDtypeStruct((N, H), F32),
            jax.ShapeDtypeStruct((N, H), F32),
            jax.ShapeDtypeStruct((N, H), F32),
        ],
    )(x, We, be, W1a, W1b, b1)


def _tc_edge(pre, rel, ea, w1c, W1d, W2, b2, Wp1, bp1, wp2, bp2):
    E, H = pre.shape
    ED = ea.shape[1]
    BE = 2000
    grid = (E // BE,)

    def body(pre_r, rel_r, ea_r, w1c_r, W1d_r, W2_r, b2_r,
             Wp1_r, bp1_r, wp2_r, bp2_r, m_r, relw_r):
        rel = rel_r[...]
        d2 = jnp.sum(rel * rel, axis=1, keepdims=True)
        z1 = (pre_r[...] + d2 * w1c_r[...]
              + jnp.dot(ea_r[...], W1d_r[...], preferred_element_type=F32))
        t = _silu(z1)
        m = _silu(jnp.dot(t, W2_r[...], preferred_element_type=F32) + b2_r[...])
        u = _silu(jnp.dot(m, Wp1_r[...], preferred_element_type=F32) + bp1_r[...])
        w = jnp.sum(u * wp2_r[...], axis=1, keepdims=True) + bp2_r[...]
        lane = lax.broadcasted_iota(jnp.int32, (1, 16), 1)
        cmark = (lane == 3).astype(F32)
        m_r[...] = m
        relw_r[...] = rel * w + cmark

    full = lambda a, b: pl.BlockSpec((a, b), lambda i: (0, 0))
    return pl.pallas_call(
        body,
        grid=grid,
        in_specs=[
            pl.BlockSpec((BE, H), lambda i: (i, 0)),
            pl.BlockSpec((BE, 16), lambda i: (i, 0)),
            pl.BlockSpec((BE, ED), lambda i: (i, 0)),
            full(1, H), full(ED, H), full(H, H), full(1, H),
            full(H, H), full(1, H), full(1, H), full(1, 1),
        ],
        out_specs=[
            pl.BlockSpec((BE, H), lambda i: (i, 0)),
            pl.BlockSpec((BE, 16), lambda i: (i, 0)),
        ],
        out_shape=[
            jax.ShapeDtypeStruct((E, H), F32),
            jax.ShapeDtypeStruct((E, 16), F32),
        ],
    )(pre, rel, ea, w1c, W1d, W2, b2, Wp1, bp1, wp2, bp2)


def _tc_node(h, pM0, pM1, pR0, pR1, posp, Wn1a, Wn1b, bn1, Wn2, bn2,
             nW1a, nW1b, nb1):
    N, H = h.shape
    BN = 2000
    grid = (N // BN,)
    has_next = nW1a is not None

    def body(*refs):
        (h_r, pM0_r, pM1_r, pR0_r, pR1_r, posp_r, Wn1a_r, Wn1b_r, bn1_r,
         Wn2_r, bn2_r) = refs[:11]
        k = 11
        if has_next:
            nW1a_r, nW1b_r, nb1_r = refs[k:k + 3]
            k += 3
        out = refs[k:]
        accM = pM0_r[...] + pM1_r[...]
        accR = pR0_r[...] + pR1_r[...]
        lane = lax.broadcasted_iota(jnp.int32, (1, 16), 1)
        cnt = jnp.sum(jnp.where(lane == 3, accR, 0.0), axis=1, keepdims=True)
        colmask = (lane < 3).astype(F32)
        upd = accR * colmask / jnp.maximum(cnt, 1.0)
        t = _silu(jnp.dot(h_r[...], Wn1a_r[...], preferred_element_type=F32)
                  + jnp.dot(accM, Wn1b_r[...], preferred_element_type=F32)
                  + bn1_r[...])
        hn = jnp.dot(t, Wn2_r[...], preferred_element_type=F32) + bn2_r[...]
        out[0][...] = hn
        out[1][...] = posp_r[...] + upd
        if has_next:
            out[2][...] = jnp.dot(hn, nW1a_r[...], preferred_element_type=F32) + nb1_r[...]
            out[3][...] = jnp.dot(hn, nW1b_r[...], preferred_element_type=F32)

    full = lambda a, b: pl.BlockSpec((a, b), lambda i: (0, 0))
    row = lambda w: pl.BlockSpec((BN, w), lambda i: (i, 0))
    in_specs = [row(H), row(H), row(H), row(16), row(16), row(16),
                full(H, H), full(H, H), full(1, H), full(H, H), full(1, H)]
    ins = [h, pM0, pM1, pR0, pR1, posp, Wn1a, Wn1b, bn1, Wn2, bn2]
    out_specs = [row(H), row(16)]
    out_shape = [jax.ShapeDtypeStruct((N, H), F32),
                 jax.ShapeDtypeStruct((N, 16), F32)]
    if has_next:
        in_specs += [full(H, H), full(H, H), full(1, H)]
        ins += [nW1a, nW1b, nb1]
        out_specs += [row(H), row(H)]
        out_shape += [jax.ShapeDtypeStruct((N, H), F32),
                      jax.ShapeDtypeStruct((N, H), F32)]
    return pl.pallas_call(
        body, grid=grid, in_specs=in_specs, out_specs=out_specs,
        out_shape=out_shape,
    )(*ins)


def _sc_gather(A, B, P, src, dst):
    N, H = A.shape
    E = src.shape[0]
    epw = E // _NW
    C = _CHUNK
    nch = epw // C
    nd = (nch - 1) // 2
    nv = H // 16

    mesh = plsc.VectorSubcoreMesh(**_MESH)

    buf_set = [
        pltpu.VMEM((C,), jnp.int32),
        pltpu.VMEM((C,), jnp.int32),
        pltpu.VMEM((C, H), F32),
        pltpu.VMEM((C, H), F32),
        pltpu.VMEM((C, 16), F32),
        pltpu.VMEM((C, 16), F32),
        pltpu.SemaphoreType.DMA,
    ]

    @functools.partial(
        pl.kernel,
        out_type=(
            jax.ShapeDtypeStruct((E, H), F32),
            jax.ShapeDtypeStruct((E, 16), F32),
        ),
        mesh=mesh,
        compiler_params=_SC_PARAMS,
        scratch_types=buf_set + buf_set,
    )
    def k(A_h, B_h, P_h, src_h, dst_h, oPre, oRel, *bufs):
        wid = lax.axis_index("s") * _NC + lax.axis_index("c")
        base0 = wid * epw
        sets = (bufs[:7], bufs[7:])

        def issue(c, bset):
            idxd, idxs, bA, bB, bPd, bPs, sem = bset
            base = base0 + c * C
            pltpu.sync_copy(dst_h.at[pl.ds(base, C)], idxd)
            pltpu.sync_copy(src_h.at[pl.ds(base, C)], idxs)
            pltpu.async_copy(A_h.at[idxd], bA, sem)
            pltpu.async_copy(B_h.at[idxs], bB, sem)
            pltpu.async_copy(P_h.at[idxd], bPd, sem)
            pltpu.async_copy(P_h.at[idxs], bPs, sem)

        def consume(c, bset):
            idxd, idxs, bA, bB, bPd, bPs, sem = bset
            base = base0 + c * C
            pltpu.make_async_copy(A_h.at[idxd], bA, sem).wait()
            pltpu.make_async_copy(B_h.at[idxs], bB, sem).wait()
            pltpu.make_async_copy(P_h.at[idxd], bPd, sem).wait()
            pltpu.make_async_copy(P_h.at[idxs], bPs, sem).wait()

            def row(i, cc):
                for kk in range(nv):
                    sl = pl.ds(kk * 16, 16)
                    bA[i, sl] = bA[i, sl] + bB[i, sl]
                bPd[i, :] = bPd[i, :] - bPs[i, :]
                return cc

            lax.fori_loop(0, C, row, 0, unroll=2)
            pltpu.sync_copy(bA, oPre.at[pl.ds(base, C)])
            pltpu.sync_copy(bPd, oRel.at[pl.ds(base, C)])

        issue(0, sets[0])
        issue(1, sets[1])

        def dstep(kk, carry):
            for b in range(2):
                c = 2 * kk + b
                consume(c, sets[b])

                @pl.when(c + 2 < nch)
                def _():
                    issue(c + 2, sets[b])

            return carry

        lax.fori_loop(0, nd, dstep, 0)
        consume(nch - 1, sets[(nch - 1) % 2])

    return k(A, B, P, src, dst)


def _sc_scatter(m, relw, dst, zM, zR):
    E, H = m.shape
    N = zM.shape[0]
    epw = E // _NW
    C = _CHUNK
    nch = epw // C
    nd = (nch - 1) // 2
    rpw = N // _NS

    mesh = plsc.VectorSubcoreMesh(**_MESH)

    buf_set = [
        pltpu.VMEM((C,), jnp.int32),
        pltpu.VMEM((C, H), F32),
        pltpu.VMEM((C, 16), F32),
        pltpu.SemaphoreType.DMA,
    ]

    @functools.partial(
        pl.kernel,
        out_type=(
            jax.ShapeDtypeStruct((_NC * N, H), F32),
            jax.ShapeDtypeStruct((_NC * N, 16), F32),
        ),
        mesh=mesh,
        compiler_params=_SC_PARAMS,
        scratch_types=buf_set + buf_set + [
            pltpu.VMEM_SHARED((N, H), F32),
            pltpu.VMEM_SHARED((N, 16), F32),
        ],
    )
    def k(m_h, relw_h, dst_h, zM_h, zR_h, oM, oR, *scr):
        sets = (scr[:4], scr[4:8])
        accM, accR = scr[8], scr[9]
        cid = lax.axis_index("c")
        sid = lax.axis_index("s")
        wid = sid * _NC + cid
        base0 = wid * epw
        r0 = sid * rpw
        pltpu.sync_copy(zM_h.at[pl.ds(r0, rpw)], accM.at[pl.ds(r0, rpw)])
        pltpu.sync_copy(zR_h.at[pl.ds(r0, rpw)], accR.at[pl.ds(r0, rpw)])
        plsc.subcore_barrier()

        def issue(c, bset):
            idx, bM, bR, sem = bset
            base = base0 + c * C
            pltpu.sync_copy(dst_h.at[pl.ds(base, C)], idx)
            pltpu.async_copy(m_h.at[pl.ds(base, C)], bM, sem)
            pltpu.async_copy(relw_h.at[pl.ds(base, C)], bR, sem)

        def consume(c, bset):
            idx, bM, bR, sem = bset
            base = base0 + c * C
            pltpu.make_async_copy(m_h.at[pl.ds(base, C)], bM, sem).wait()
            pltpu.make_async_copy(relw_h.at[pl.ds(base, C)], bR, sem).wait()
            pltpu.sync_copy(bM, accM.at[idx], add=True)
            pltpu.sync_copy(bR, accR.at[idx], add=True)

        issue(0, sets[0])
        issue(1, sets[1])

        def dstep(kk, carry):
            for b in range(2):
                c = 2 * kk + b
                consume(c, sets[b])

                @pl.when(c + 2 < nch)
                def _():
                    issue(c + 2, sets[b])

            return carry

        lax.fori_loop(0, nd, dstep, 0)
        consume(nch - 1, sets[(nch - 1) % 2])
        plsc.subcore_barrier()
        pltpu.sync_copy(accM.at[pl.ds(r0, rpw)], oM.at[pl.ds(cid * N + r0, rpw)])
        pltpu.sync_copy(accR.at[pl.ds(r0, rpw)], oR.at[pl.ds(cid * N + r0, rpw)])

    return k(m, relw, dst, zM, zR)


def kernel(edge_index, x, pos, edge_attr, params):
    src = edge_index[0]
    dst = edge_index[1]
    N = x.shape[0]
    H = params["emb"]["W"][0].shape[1]

    posp = jnp.pad(pos, ((0, 0), (0, 16 - pos.shape[1])))
    zM = jnp.zeros((N, H), F32)
    zR = jnp.zeros((N, 16), F32)

    layers = params["layers"]
    emb = params["emb"]

    def edge_w(lp):
        W1 = lp["edge"]["W"][0]
        return (W1[:H], W1[H:2 * H], W1[2 * H:2 * H + 1],
                W1[2 * H + 1:], lp["edge"]["b"][0].reshape(1, H))

    W1a0, W1b0, _, _, b10 = edge_w(layers[0])
    h, A, B = _tc_embed(x, emb["W"][0], emb["b"][0].reshape(1, H),
                        W1a0, W1b0, b10)

    for li, lp in enumerate(layers):
        _, _, w1c, W1d, _ = edge_w(lp)
        pre, rel = _sc_gather(A, B, posp, src, dst)
        m, relw = _tc_edge(
            pre, rel, edge_attr,
            w1c, W1d,
            lp["edge"]["W"][1], lp["edge"]["b"][1].reshape(1, H),
            lp["pos"]["W"][0], lp["pos"]["b"][0].reshape(1, H),
            lp["pos"]["W"][1].reshape(1, H), lp["pos"]["b"][1].reshape(1, 1),
        )
        pM, pR = _sc_scatter(m, relw, dst, zM, zR)
        Wn1 = lp["node"]["W"][0]
        is_last = li == len(layers) - 1
        if not is_last:
            nW1a, nW1b, _, _, nb1 = edge_w(layers[li + 1])
        else:
            nW1a = nW1b = nb1 = None
        outs = _tc_node(
            h, pM[:N], pM[N:], pR[:N], pR[N:], posp,
            Wn1[:H], Wn1[H:], lp["node"]["b"][0].reshape(1, H),
            lp["node"]["W"][1], lp["node"]["b"][1].reshape(1, H),
            nW1a, nW1b, nb1,
        )
        if not is_last:
            h, posp, A, B = outs
        else:
            h, posp = outs

    return h, posp[:, :3]


# speedup vs baseline: 3.9425x; 1.2809x over previous
import functools

import jax
import jax.numpy as jnp
from jax import lax
from jax.experimental import pallas as pl
from jax.experimental.pallas import tpu as pltpu
from jax.experimental.pallas import tpu_sc as plsc

F32 = jnp.float32

_NC = 2
_NS = 16
_NW = _NC * _NS

_CHUNK = 80

_TW = 80

_MESH = dict(core_axis_name="c", subcore_axis_name="s")
_SC_PARAMS = pltpu.CompilerParams(use_tc_tiling_on_sc=False)


def _silu(v):
    return v * jax.nn.sigmoid(v)


def _tc_embed(x, posp, We, be, W1a, W1b, b1):
    N, IC = x.shape
    H = We.shape[1]
    BN = 2000
    grid = (N // BN,)

    def body(x_r, posp_r, We_r, be_r, W1a_r, W1b_r, b1_r, h_r, TA_r, TB_r):
        h = jnp.dot(x_r[...], We_r[...], preferred_element_type=F32) + be_r[...]
        h_r[...] = h
        pp = posp_r[...]
        A = jnp.dot(h, W1a_r[...], preferred_element_type=F32) + b1_r[...]
        B = jnp.dot(h, W1b_r[...], preferred_element_type=F32)
        TA_r[...] = jnp.concatenate([A, pp], axis=1)
        TB_r[...] = jnp.concatenate([B, pp], axis=1)

    full = lambda a, b: pl.BlockSpec((a, b), lambda i: (0, 0))
    row = lambda w: pl.BlockSpec((BN, w), lambda i: (i, 0))
    return pl.pallas_call(
        body,
        grid=grid,
        in_specs=[
            row(IC), row(16),
            full(IC, H), full(1, H), full(H, H), full(H, H), full(1, H),
        ],
        out_specs=[row(H), row(_TW), row(_TW)],
        out_shape=[
            jax.ShapeDtypeStruct((N, H), F32),
            jax.ShapeDtypeStruct((N, _TW), F32),
            jax.ShapeDtypeStruct((N, _TW), F32),
        ],
    )(x, posp, We, be, W1a, W1b, b1)


def _tc_edge(gA, gB, ea, w1c, W1d, W2, b2, Wp1, bp1, wp2, bp2):
    E = gA.shape[0]
    H = _TW - 16
    ED = ea.shape[1]
    BE = 2000
    grid = (E // BE,)

    def body(gA_r, gB_r, ea_r, w1c_r, W1d_r, W2_r, b2_r,
             Wp1_r, bp1_r, wp2_r, bp2_r, msg_r):
        a = gA_r[...]
        b = gB_r[...]
        pre = a[:, :H] + b[:, :H]
        rel = a[:, H:] - b[:, H:]
        d2 = jnp.sum(rel * rel, axis=1, keepdims=True)
        z1 = (pre + d2 * w1c_r[...]
              + jnp.dot(ea_r[...], W1d_r[...], preferred_element_type=F32))
        t = _silu(z1)
        m = _silu(jnp.dot(t, W2_r[...], preferred_element_type=F32) + b2_r[...])
        u = _silu(jnp.dot(m, Wp1_r[...], preferred_element_type=F32) + bp1_r[...])
        w = jnp.sum(u * wp2_r[...], axis=1, keepdims=True) + bp2_r[...]
        lane = lax.broadcasted_iota(jnp.int32, (1, 16), 1)
        cmark = (lane == 3).astype(F32)
        msg_r[...] = jnp.concatenate([m, rel * w + cmark], axis=1)

    full = lambda a, b: pl.BlockSpec((a, b), lambda i: (0, 0))
    return pl.pallas_call(
        body,
        grid=grid,
        in_specs=[
            pl.BlockSpec((BE, _TW), lambda i: (i, 0)),
            pl.BlockSpec((BE, _TW), lambda i: (i, 0)),
            pl.BlockSpec((BE, ED), lambda i: (i, 0)),
            full(1, H), full(ED, H), full(H, H), full(1, H),
            full(H, H), full(1, H), full(1, H), full(1, 1),
        ],
        out_specs=[pl.BlockSpec((BE, _TW), lambda i: (i, 0))],
        out_shape=[jax.ShapeDtypeStruct((E, _TW), F32)],
    )(gA, gB, ea, w1c, W1d, W2, b2, Wp1, bp1, wp2, bp2)[0]


def _tc_node(h, p0, p1, posp, Wn1a, Wn1b, bn1, Wn2, bn2, nW1a, nW1b, nb1):
    N, H = h.shape
    BN = 2000
    grid = (N // BN,)
    has_next = nW1a is not None

    def body(*refs):
        (h_r, p0_r, p1_r, posp_r, Wn1a_r, Wn1b_r, bn1_r,
         Wn2_r, bn2_r) = refs[:9]
        k = 9
        if has_next:
            nW1a_r, nW1b_r, nb1_r = refs[k:k + 3]
            k += 3
        out = refs[k:]
        acc = p0_r[...] + p1_r[...]
        accM = acc[:, :H]
        accR = acc[:, H:]
        lane = lax.broadcasted_iota(jnp.int32, (1, 16), 1)
        cnt = jnp.sum(jnp.where(lane == 3, accR, 0.0), axis=1, keepdims=True)
        colmask = (lane < 3).astype(F32)
        upd = accR * colmask / jnp.maximum(cnt, 1.0)
        pp = posp_r[...] + upd
        t = _silu(jnp.dot(h_r[...], Wn1a_r[...], preferred_element_type=F32)
                  + jnp.dot(accM, Wn1b_r[...], preferred_element_type=F32)
                  + bn1_r[...])
        hn = jnp.dot(t, Wn2_r[...], preferred_element_type=F32) + bn2_r[...]
        out[0][...] = hn
        out[1][...] = pp
        if has_next:
            A = jnp.dot(hn, nW1a_r[...], preferred_element_type=F32) + nb1_r[...]
            B = jnp.dot(hn, nW1b_r[...], preferred_element_type=F32)
            out[2][...] = jnp.concatenate([A, pp], axis=1)
            out[3][...] = jnp.concatenate([B, pp], axis=1)

    full = lambda a, b: pl.BlockSpec((a, b), lambda i: (0, 0))
    row = lambda w: pl.BlockSpec((BN, w), lambda i: (i, 0))
    in_specs = [row(H), row(_TW), row(_TW), row(16),
                full(H, H), full(H, H), full(1, H), full(H, H), full(1, H)]
    ins = [h, p0, p1, posp, Wn1a, Wn1b, bn1, Wn2, bn2]
    out_specs = [row(H), row(16)]
    out_shape = [jax.ShapeDtypeStruct((N, H), F32),
                 jax.ShapeDtypeStruct((N, 16), F32)]
    if has_next:
        in_specs += [full(H, H), full(H, H), full(1, H)]
        ins += [nW1a, nW1b, nb1]
        out_specs += [row(_TW), row(_TW)]
        out_shape += [jax.ShapeDtypeStruct((N, _TW), F32),
                      jax.ShapeDtypeStruct((N, _TW), F32)]
    return pl.pallas_call(
        body, grid=grid, in_specs=in_specs, out_specs=out_specs,
        out_shape=out_shape,
    )(*ins)


def _sc_gather(TA, TB, src2, dst2):
    N = TA.shape[0]
    nrow, C = src2.shape
    E = nrow * C
    epw = E // _NW
    nch = epw // C
    NB = 4
    nq = nch // NB
    rem = nch - nq * NB

    mesh = plsc.VectorSubcoreMesh(**_MESH)

    buf_set = [
        pltpu.VMEM((C, _TW), F32),
        pltpu.VMEM((C, _TW), F32),
        pltpu.SemaphoreType.DMA,
        pltpu.SemaphoreType.DMA,
    ]

    @functools.partial(
        pl.kernel,
        out_type=(
            jax.ShapeDtypeStruct((E, _TW), F32),
            jax.ShapeDtypeStruct((E, _TW), F32),
        ),
        mesh=mesh,
        compiler_params=_SC_PARAMS,
        scratch_types=buf_set * NB + [
            pltpu.VMEM((nch, C), jnp.int32),
            pltpu.VMEM((nch, C), jnp.int32),
        ],
    )
    def k(TA_h, TB_h, src_h, dst_h, oA, oB, *scr):
        sets = tuple(scr[4 * i:4 * i + 4] for i in range(NB))
        idxd, idxs = scr[4 * NB], scr[4 * NB + 1]
        wid = lax.axis_index("s") * _NC + lax.axis_index("c")
        base0 = wid * epw
        row0 = wid * nch
        pltpu.sync_copy(dst_h.at[pl.ds(row0, nch)], idxd)
        pltpu.sync_copy(src_h.at[pl.ds(row0, nch)], idxs)

        def issue(c, bset):
            bA, bB, gsem, _ = bset
            pltpu.async_copy(TA_h.at[idxd.at[c]], bA, gsem)
            pltpu.async_copy(TB_h.at[idxs.at[c]], bB, gsem)

        def gwait(c, bset):
            bA, bB, gsem, _ = bset
            pltpu.make_async_copy(TA_h.at[idxd.at[c]], bA, gsem).wait()
            pltpu.make_async_copy(TB_h.at[idxs.at[c]], bB, gsem).wait()

        def wdrain(bset):
            bA, bB, _, wsem = bset
            pltpu.make_async_copy(bA, oA.at[pl.ds(base0, C)], wsem).wait()
            pltpu.make_async_copy(bB, oB.at[pl.ds(base0, C)], wsem).wait()

        def writeback(c, bset):
            bA, bB, _, wsem = bset
            base = base0 + c * C
            pltpu.async_copy(bA, oA.at[pl.ds(base, C)], wsem)
            pltpu.async_copy(bB, oB.at[pl.ds(base, C)], wsem)

        def step(c, b):
            gwait(c, sets[b])
            writeback(c, sets[b])

            @pl.when(c >= 2)
            def _():
                wdrain(sets[(b + 2) % NB])

            @pl.when(c + 2 < nch)
            def _():
                issue(c + 2, sets[(b + 2) % NB])

        issue(0, sets[0])
        issue(1, sets[1])

        def qstep(kk, carry):
            for b in range(NB):
                step(NB * kk + b, b)
            return carry

        lax.fori_loop(0, nq, qstep, 0)
        for b in range(rem):
            step(nq * NB + b, b)
        wdrain(sets[(nch - 2) % NB])
        wdrain(sets[(nch - 1) % NB])

    return k(TA, TB, src2, dst2)


def _sc_scatter(msg, dst2, zP):
    N = zP.shape[0]
    nrow, C = dst2.shape
    E = nrow * C
    epw = E // _NW
    nch = epw // C
    nd = (nch - 1) // 2
    rpw = N // _NS

    mesh = plsc.VectorSubcoreMesh(**_MESH)

    buf_set = [
        pltpu.VMEM((C, _TW), F32),
        pltpu.SemaphoreType.DMA,
    ]

    @functools.partial(
        pl.kernel,
        out_type=jax.ShapeDtypeStruct((_NC * N, _TW), F32),
        mesh=mesh,
        compiler_params=_SC_PARAMS,
        scratch_types=buf_set + buf_set + [
            pltpu.VMEM((nch, C), jnp.int32),
            pltpu.VMEM_SHARED((N, _TW), F32),
        ],
    )
    def k(msg_h, dst_h, zP_h, oP, bM0, sem0, bM1, sem1, idx, acc):
        sets = ((bM0, sem0), (bM1, sem1))
        cid = lax.axis_index("c")
        sid = lax.axis_index("s")
        wid = sid * _NC + cid
        base0 = wid * epw
        row0 = wid * nch
        r0 = sid * rpw
        pltpu.sync_copy(zP_h.at[pl.ds(r0, rpw)], acc.at[pl.ds(r0, rpw)])
        pltpu.sync_copy(dst_h.at[pl.ds(row0, nch)], idx)
        plsc.subcore_barrier()

        def issue(c, bset):
            bM, sem = bset
            pltpu.async_copy(msg_h.at[pl.ds(base0 + c * C, C)], bM, sem)

        def consume(c, bset):
            bM, sem = bset
            pltpu.make_async_copy(msg_h.at[pl.ds(base0 + c * C, C)], bM, sem).wait()
            pltpu.sync_copy(bM, acc.at[idx.at[c]], add=True)

        issue(0, sets[0])
        issue(1, sets[1])

        def dstep(kk, carry):
            for b in range(2):
                c = 2 * kk + b
                consume(c, sets[b])

                @pl.when(c + 2 < nch)
                def _():
                    issue(c + 2, sets[b])

            return carry

        lax.fori_loop(0, nd, dstep, 0)
        consume(nch - 1, sets[(nch - 1) % 2])
        plsc.subcore_barrier()
        pltpu.sync_copy(acc.at[pl.ds(r0, rpw)], oP.at[pl.ds(cid * N + r0, rpw)])

    return k(msg, dst2, zP)


def kernel(edge_index, x, pos, edge_attr, params):
    src = edge_index[0]
    dst = edge_index[1]
    N = x.shape[0]
    E = src.shape[0]
    H = params["emb"]["W"][0].shape[1]

    posp = jnp.pad(pos, ((0, 0), (0, 16 - pos.shape[1])))
    zP = jnp.zeros((N, _TW), F32)
    src2 = src.reshape(E // _CHUNK, _CHUNK)
    dst2 = dst.reshape(E // _CHUNK, _CHUNK)

    layers = params["layers"]
    emb = params["emb"]

    def edge_w(lp):
        W1 = lp["edge"]["W"][0]
        return (W1[:H], W1[H:2 * H], W1[2 * H:2 * H + 1],
                W1[2 * H + 1:], lp["edge"]["b"][0].reshape(1, H))

    W1a0, W1b0, _, _, b10 = edge_w(layers[0])
    h, TA, TB = _tc_embed(x, posp, emb["W"][0], emb["b"][0].reshape(1, H),
                          W1a0, W1b0, b10)

    for li, lp in enumerate(layers):
        _, _, w1c, W1d, _ = edge_w(lp)
        gA, gB = _sc_gather(TA, TB, src2, dst2)
        msg = _tc_edge(
            gA, gB, edge_attr,
            w1c, W1d,
            lp["edge"]["W"][1], lp["edge"]["b"][1].reshape(1, H),
            lp["pos"]["W"][0], lp["pos"]["b"][0].reshape(1, H),
            lp["pos"]["W"][1].reshape(1, H), lp["pos"]["b"][1].reshape(1, 1),
        )
        part = _sc_scatter(msg, dst2, zP)
        Wn1 = lp["node"]["W"][0]
        is_last = li == len(layers) - 1
        if not is_last:
            nW1a, nW1b, _, _, nb1 = edge_w(layers[li + 1])
        else:
            nW1a = nW1b = nb1 = None
        outs = _tc_node(
            h, part[:N], part[N:], posp,
            Wn1[:H], Wn1[H:], lp["node"]["b"][0].reshape(1, H),
            lp["node"]["W"][1], lp["node"]["b"][1].reshape(1, H),
            nW1a, nW1b, nb1,
        )
        if not is_last:
            h, posp, TA, TB = outs
        else:
            h, posp = outs

    return h, posp[:, :3]


# speedup vs baseline: 4.0523x; 1.0278x over previous
import functools

import jax
import jax.numpy as jnp
from jax import lax
from jax.experimental import pallas as pl
from jax.experimental.pallas import tpu as pltpu
from jax.experimental.pallas import tpu_sc as plsc

F32 = jnp.float32

_NC = 2
_NS = 16
_NW = _NC * _NS

_CHUNK = 80

_TW = 80

_MESH = dict(core_axis_name="c", subcore_axis_name="s")
_SC_PARAMS = pltpu.CompilerParams(use_tc_tiling_on_sc=False)


def _silu(v):
    return v * jax.nn.sigmoid(v)


def _tc_embed(x, posp, We, be, W1a, W1b, b1):
    N, IC = x.shape
    H = We.shape[1]
    BN = 2000
    grid = (N // BN,)

    def body(x_r, posp_r, We_r, be_r, W1a_r, W1b_r, b1_r, h_r, TA_r, TB_r):
        h = jnp.dot(x_r[...], We_r[...], preferred_element_type=F32) + be_r[...]
        h_r[...] = h
        pp = posp_r[...]
        A = jnp.dot(h, W1a_r[...], preferred_element_type=F32) + b1_r[...]
        B = jnp.dot(h, W1b_r[...], preferred_element_type=F32)
        TA_r[...] = jnp.concatenate([A, pp], axis=1)
        TB_r[...] = jnp.concatenate([B, pp], axis=1)

    full = lambda a, b: pl.BlockSpec((a, b), lambda i: (0, 0))
    row = lambda w: pl.BlockSpec((BN, w), lambda i: (i, 0))
    return pl.pallas_call(
        body,
        grid=grid,
        in_specs=[
            row(IC), row(16),
            full(IC, H), full(1, H), full(H, H), full(H, H), full(1, H),
        ],
        out_specs=[row(H), row(_TW), row(_TW)],
        out_shape=[
            jax.ShapeDtypeStruct((N, H), F32),
            jax.ShapeDtypeStruct((N, _TW), F32),
            jax.ShapeDtypeStruct((N, _TW), F32),
        ],
    )(x, posp, We, be, W1a, W1b, b1)


def _tc_edge(gA, gB, ea, w1c, W1d, W2, b2, Wp1, bp1, wp2, bp2):
    E = gA.shape[0]
    H = _TW - 16
    ED = ea.shape[1]
    BE = 2000
    grid = (E // BE,)

    def body(gA_r, gB_r, ea_r, w1c_r, W1d_r, W2_r, b2_r,
             Wp1_r, bp1_r, wp2_r, bp2_r, msg_r):
        a = gA_r[...]
        b = gB_r[...]
        pre = a[:, :H] + b[:, :H]
        rel = a[:, H:] - b[:, H:]
        d2 = jnp.sum(rel * rel, axis=1, keepdims=True)
        z1 = (pre + d2 * w1c_r[...]
              + jnp.dot(ea_r[...], W1d_r[...], preferred_element_type=F32))
        t = _silu(z1)
        m = _silu(jnp.dot(t, W2_r[...], preferred_element_type=F32) + b2_r[...])
        u = _silu(jnp.dot(m, Wp1_r[...], preferred_element_type=F32) + bp1_r[...])
        w = jnp.sum(u * wp2_r[...], axis=1, keepdims=True) + bp2_r[...]
        lane = lax.broadcasted_iota(jnp.int32, (1, 16), 1)
        cmark = (lane == 3).astype(F32)
        msg_r[...] = jnp.concatenate([m, rel * w + cmark], axis=1)

    full = lambda a, b: pl.BlockSpec((a, b), lambda i: (0, 0))
    return pl.pallas_call(
        body,
        grid=grid,
        in_specs=[
            pl.BlockSpec((BE, _TW), lambda i: (i, 0)),
            pl.BlockSpec((BE, _TW), lambda i: (i, 0)),
            pl.BlockSpec((BE, ED), lambda i: (i, 0)),
            full(1, H), full(ED, H), full(H, H), full(1, H),
            full(H, H), full(1, H), full(1, H), full(1, 1),
        ],
        out_specs=[pl.BlockSpec((BE, _TW), lambda i: (i, 0))],
        out_shape=[jax.ShapeDtypeStruct((E, _TW), F32)],
    )(gA, gB, ea, w1c, W1d, W2, b2, Wp1, bp1, wp2, bp2)[0]


def _tc_node(h, parts, posp, Wn1a, Wn1b, bn1, Wn2, bn2, nW1a, nW1b, nb1):
    N, H = h.shape
    BN = 2000
    grid = (N // BN,)
    has_next = nW1a is not None
    np_ = len(parts)

    def body(*refs):
        h_r = refs[0]
        part_rs = refs[1:1 + np_]
        (posp_r, Wn1a_r, Wn1b_r, bn1_r, Wn2_r, bn2_r) = refs[1 + np_:7 + np_]
        k = 7 + np_
        if has_next:
            nW1a_r, nW1b_r, nb1_r = refs[k:k + 3]
            k += 3
        out = refs[k:]
        acc = part_rs[0][...]
        for pr in part_rs[1:]:
            acc = acc + pr[...]
        accM = acc[:, :H]
        accR = acc[:, H:]
        lane = lax.broadcasted_iota(jnp.int32, (1, 16), 1)
        cnt = jnp.sum(jnp.where(lane == 3, accR, 0.0), axis=1, keepdims=True)
        colmask = (lane < 3).astype(F32)
        upd = accR * colmask / jnp.maximum(cnt, 1.0)
        pp = posp_r[...] + upd
        t = _silu(jnp.dot(h_r[...], Wn1a_r[...], preferred_element_type=F32)
                  + jnp.dot(accM, Wn1b_r[...], preferred_element_type=F32)
                  + bn1_r[...])
        hn = jnp.dot(t, Wn2_r[...], preferred_element_type=F32) + bn2_r[...]
        out[0][...] = hn
        out[1][...] = pp
        if has_next:
            A = jnp.dot(hn, nW1a_r[...], preferred_element_type=F32) + nb1_r[...]
            B = jnp.dot(hn, nW1b_r[...], preferred_element_type=F32)
            out[2][...] = jnp.concatenate([A, pp], axis=1)
            out[3][...] = jnp.concatenate([B, pp], axis=1)

    full = lambda a, b: pl.BlockSpec((a, b), lambda i: (0, 0))
    row = lambda w: pl.BlockSpec((BN, w), lambda i: (i, 0))
    in_specs = [row(H)] + [row(_TW)] * np_ + [row(16),
                full(H, H), full(H, H), full(1, H), full(H, H), full(1, H)]
    ins = [h] + list(parts) + [posp, Wn1a, Wn1b, bn1, Wn2, bn2]
    out_specs = [row(H), row(16)]
    out_shape = [jax.ShapeDtypeStruct((N, H), F32),
                 jax.ShapeDtypeStruct((N, 16), F32)]
    if has_next:
        in_specs += [full(H, H), full(H, H), full(1, H)]
        ins += [nW1a, nW1b, nb1]
        out_specs += [row(_TW), row(_TW)]
        out_shape += [jax.ShapeDtypeStruct((N, _TW), F32),
                      jax.ShapeDtypeStruct((N, _TW), F32)]
    return pl.pallas_call(
        body, grid=grid, in_specs=in_specs, out_specs=out_specs,
        out_shape=out_shape,
    )(*ins)


def _sc_gather(TA, TB, src2, dst2):
    N = TA.shape[0]
    nrow, C = src2.shape
    E = nrow * C
    epw = E // _NW
    nch = epw // C
    NB = 4
    nq = nch // NB
    rem = nch - nq * NB

    mesh = plsc.VectorSubcoreMesh(**_MESH)

    buf_set = [
        pltpu.VMEM((C, _TW), F32),
        pltpu.VMEM((C, _TW), F32),
        pltpu.SemaphoreType.DMA,
        pltpu.SemaphoreType.DMA,
    ]

    @functools.partial(
        pl.kernel,
        out_type=(
            jax.ShapeDtypeStruct((E, _TW), F32),
            jax.ShapeDtypeStruct((E, _TW), F32),
        ),
        mesh=mesh,
        compiler_params=_SC_PARAMS,
        scratch_types=buf_set * NB + [
            pltpu.VMEM((nch, C), jnp.int32),
            pltpu.VMEM((nch, C), jnp.int32),
        ],
    )
    def k(TA_h, TB_h, src_h, dst_h, oA, oB, *scr):
        sets = tuple(scr[4 * i:4 * i + 4] for i in range(NB))
        idxd, idxs = scr[4 * NB], scr[4 * NB + 1]
        wid = lax.axis_index("s") * _NC + lax.axis_index("c")
        base0 = wid * epw
        row0 = wid * nch
        pltpu.sync_copy(dst_h.at[pl.ds(row0, nch)], idxd)
        pltpu.sync_copy(src_h.at[pl.ds(row0, nch)], idxs)

        def issue(c, bset):
            bA, bB, gsem, _ = bset
            pltpu.async_copy(TA_h.at[idxd.at[c]], bA, gsem)
            pltpu.async_copy(TB_h.at[idxs.at[c]], bB, gsem)

        def gwait(c, bset):
            bA, bB, gsem, _ = bset
            pltpu.make_async_copy(TA_h.at[idxd.at[c]], bA, gsem).wait()
            pltpu.make_async_copy(TB_h.at[idxs.at[c]], bB, gsem).wait()

        def wdrain(bset):
            bA, bB, _, wsem = bset
            pltpu.make_async_copy(bA, oA.at[pl.ds(base0, C)], wsem).wait()
            pltpu.make_async_copy(bB, oB.at[pl.ds(base0, C)], wsem).wait()

        def writeback(c, bset):
            bA, bB, _, wsem = bset
            base = base0 + c * C
            pltpu.async_copy(bA, oA.at[pl.ds(base, C)], wsem)
            pltpu.async_copy(bB, oB.at[pl.ds(base, C)], wsem)

        def step(c, b):
            gwait(c, sets[b])
            writeback(c, sets[b])

            @pl.when(c >= 2)
            def _():
                wdrain(sets[(b + 2) % NB])

            @pl.when(c + 2 < nch)
            def _():
                issue(c + 2, sets[(b + 2) % NB])

        issue(0, sets[0])
        issue(1, sets[1])

        def qstep(kk, carry):
            for b in range(NB):
                step(NB * kk + b, b)
            return carry

        lax.fori_loop(0, nq, qstep, 0)
        for b in range(rem):
            step(nq * NB + b, b)
        wdrain(sets[(nch - 2) % NB])
        wdrain(sets[(nch - 1) % NB])

    return k(TA, TB, src2, dst2)


def _sc_scatter(msg, dst2, zP):
    N = zP.shape[0]
    nrow, C = dst2.shape
    E = nrow * C
    epw = E // _NW
    nch = epw // C
    nq = nch // 2
    rem = nch - nq * 2
    rpw = N // _NS

    mesh = plsc.VectorSubcoreMesh(**_MESH)

    buf_set = [
        pltpu.VMEM((C, _TW), F32),
        pltpu.SemaphoreType.DMA,
    ]

    @functools.partial(
        pl.kernel,
        out_type=jax.ShapeDtypeStruct((_NC * N, _TW), F32),
        mesh=mesh,
        compiler_params=_SC_PARAMS,
        scratch_types=buf_set + buf_set + [
            pltpu.VMEM((nch, C), jnp.int32),
            pltpu.VMEM_SHARED((N, _TW), F32),
        ],
    )
    def k(msg_h, dst_h, zP_h, oP, bM0, sem0, bM1, sem1, idx, acc):
        sets = ((bM0, sem0), (bM1, sem1))
        cid = lax.axis_index("c")
        sid = lax.axis_index("s")
        wid = sid * _NC + cid
        base0 = wid * epw
        row0 = wid * nch
        r0 = sid * rpw
        pltpu.sync_copy(zP_h.at[pl.ds(r0, rpw)], acc.at[pl.ds(r0, rpw)])
        pltpu.sync_copy(dst_h.at[pl.ds(row0, nch)], idx)
        plsc.subcore_barrier()

        def issue(c, bset):
            bM, sem = bset
            pltpu.async_copy(msg_h.at[pl.ds(base0 + c * C, C)], bM, sem)

        def consume(c, bset):
            bM, sem = bset
            pltpu.make_async_copy(msg_h.at[pl.ds(base0 + c * C, C)], bM, sem).wait()
            pltpu.sync_copy(bM, acc.at[idx.at[c]], add=True)

        issue(0, sets[0])
        issue(1, sets[1])

        def dstep(kk, carry):
            for b in range(2):
                c = 2 * kk + b
                consume(c, sets[b])

                @pl.when(c + 2 < nch)
                def _():
                    issue(c + 2, sets[b])

            return carry

        lax.fori_loop(0, nq, dstep, 0)
        for b in range(rem):
            consume(nq * 2 + b, sets[b])
        plsc.subcore_barrier()
        pltpu.sync_copy(acc.at[pl.ds(r0, rpw)], oP.at[pl.ds(cid * N + r0, rpw)])

    return k(msg, dst2, zP)


def kernel(edge_index, x, pos, edge_attr, params):
    src = edge_index[0]
    dst = edge_index[1]
    N = x.shape[0]
    E = src.shape[0]
    H = params["emb"]["W"][0].shape[1]

    posp = jnp.pad(pos, ((0, 0), (0, 16 - pos.shape[1])))
    zP = jnp.zeros((N, _TW), F32)
    src2 = src.reshape(E // _CHUNK, _CHUNK)
    dst2 = dst.reshape(E // _CHUNK, _CHUNK)
    E1 = (E * 3 // 5) // (_NW * _CHUNK) * (_NW * _CHUNK)
    r1 = E1 // _CHUNK
    splits = ((src2[:r1], dst2[:r1], edge_attr[:E1]),
              (src2[r1:], dst2[r1:], edge_attr[E1:]))

    layers = params["layers"]
    emb = params["emb"]

    def edge_w(lp):
        W1 = lp["edge"]["W"][0]
        return (W1[:H], W1[H:2 * H], W1[2 * H:2 * H + 1],
                W1[2 * H + 1:], lp["edge"]["b"][0].reshape(1, H))

    W1a0, W1b0, _, _, b10 = edge_w(layers[0])
    h, TA, TB = _tc_embed(x, posp, emb["W"][0], emb["b"][0].reshape(1, H),
                          W1a0, W1b0, b10)

    for li, lp in enumerate(layers):
        _, _, w1c, W1d, _ = edge_w(lp)
        ew = (w1c, W1d,
              lp["edge"]["W"][1], lp["edge"]["b"][1].reshape(1, H),
              lp["pos"]["W"][0], lp["pos"]["b"][0].reshape(1, H),
              lp["pos"]["W"][1].reshape(1, H), lp["pos"]["b"][1].reshape(1, 1))
        g1 = _sc_gather(TA, TB, splits[0][0], splits[0][1])
        g2 = _sc_gather(TA, TB, splits[1][0], splits[1][1])
        msg1 = _tc_edge(g1[0], g1[1], splits[0][2], *ew)
        part1 = _sc_scatter(msg1, splits[0][1], zP)
        msg2 = _tc_edge(g2[0], g2[1], splits[1][2], *ew)
        part2 = _sc_scatter(msg2, splits[1][1], zP)
        Wn1 = lp["node"]["W"][0]
        is_last = li == len(layers) - 1
        if not is_last:
            nW1a, nW1b, _, _, nb1 = edge_w(layers[li + 1])
        else:
            nW1a = nW1b = nb1 = None
        outs = _tc_node(
            h, (part1[:N], part1[N:], part2[:N], part2[N:]), posp,
            Wn1[:H], Wn1[H:], lp["node"]["b"][0].reshape(1, H),
            lp["node"]["W"][1], lp["node"]["b"][1].reshape(1, H),
            nW1a, nW1b, nb1,
        )
        if not is_last:
            h, posp, TA, TB = outs
        else:
            h, posp = outs

    return h, posp[:, :3]


# speedup vs baseline: 4.0540x; 1.0004x over previous
import functools

import jax
import jax.numpy as jnp
from jax import lax
from jax.experimental import pallas as pl
from jax.experimental.pallas import tpu as pltpu
from jax.experimental.pallas import tpu_sc as plsc

F32 = jnp.float32

_NC = 2
_NS = 16
_NW = _NC * _NS

_CHUNK = 80

_TW = 80

_MESH = dict(core_axis_name="c", subcore_axis_name="s")
_SC_PARAMS = pltpu.CompilerParams(use_tc_tiling_on_sc=False)


def _silu(v):
    return v * jax.nn.sigmoid(v)


def _dot(a, b, preferred_element_type=F32):
    return jnp.dot(a, b, preferred_element_type=preferred_element_type)


def _tc_embed(x, posp, We, be, W1a, W1b, b1):
    N, IC = x.shape
    H = We.shape[1]
    BN = 2000
    grid = (N // BN,)

    def body(x_r, posp_r, We_r, be_r, W1a_r, W1b_r, b1_r, h_r, TA_r, TB_r):
        h = _dot(x_r[...], We_r[...], preferred_element_type=F32) + be_r[...]
        h_r[...] = h
        pp = posp_r[...]
        A = _dot(h, W1a_r[...], preferred_element_type=F32) + b1_r[...]
        B = _dot(h, W1b_r[...], preferred_element_type=F32)
        TA_r[...] = jnp.concatenate([A, pp], axis=1)
        TB_r[...] = jnp.concatenate([B, pp], axis=1)

    full = lambda a, b: pl.BlockSpec((a, b), lambda i: (0, 0))
    row = lambda w: pl.BlockSpec((BN, w), lambda i: (i, 0))
    return pl.pallas_call(
        body,
        grid=grid,
        in_specs=[
            row(IC), row(16),
            full(IC, H), full(1, H), full(H, H), full(H, H), full(1, H),
        ],
        out_specs=[row(H), row(_TW), row(_TW)],
        out_shape=[
            jax.ShapeDtypeStruct((N, H), F32),
            jax.ShapeDtypeStruct((N, _TW), F32),
            jax.ShapeDtypeStruct((N, _TW), F32),
        ],
    )(x, posp, We, be, W1a, W1b, b1)


def _tc_edge(gA, gB, ea, w1c, W1d, W2, b2, Wp1, bp1, wp2, bp2):
    E = gA.shape[0]
    H = _TW - 16
    ED = ea.shape[1]
    BE = 2000
    grid = (E // BE,)

    def body(gA_r, gB_r, ea_r, w1c_r, W1d_r, W2_r, b2_r,
             Wp1_r, bp1_r, wp2_r, bp2_r, msg_r):
        a = gA_r[...]
        b = gB_r[...]
        pre = a[:, :H] + b[:, :H]
        rel = a[:, H:] - b[:, H:]
        d2 = jnp.sum(rel * rel, axis=1, keepdims=True)
        z1 = (pre + d2 * w1c_r[...]
              + _dot(ea_r[...], W1d_r[...], preferred_element_type=F32))
        t = _silu(z1)
        m = _silu(_dot(t, W2_r[...], preferred_element_type=F32) + b2_r[...])
        u = _silu(_dot(m, Wp1_r[...], preferred_element_type=F32) + bp1_r[...])
        w = jnp.sum(u * wp2_r[...], axis=1, keepdims=True) + bp2_r[...]
        lane = lax.broadcasted_iota(jnp.int32, (1, 16), 1)
        cmark = (lane == 3).astype(F32)
        msg_r[...] = jnp.concatenate([m, rel * w + cmark], axis=1)

    full = lambda a, b: pl.BlockSpec((a, b), lambda i: (0, 0))
    return pl.pallas_call(
        body,
        grid=grid,
        in_specs=[
            pl.BlockSpec((BE, _TW), lambda i: (i, 0)),
            pl.BlockSpec((BE, _TW), lambda i: (i, 0)),
            pl.BlockSpec((BE, ED), lambda i: (i, 0)),
            full(1, H), full(ED, H), full(H, H), full(1, H),
            full(H, H), full(1, H), full(1, H), full(1, 1),
        ],
        out_specs=[pl.BlockSpec((BE, _TW), lambda i: (i, 0))],
        out_shape=[jax.ShapeDtypeStruct((E, _TW), F32)],
    )(gA, gB, ea, w1c, W1d, W2, b2, Wp1, bp1, wp2, bp2)[0]


def _tc_node(h, parts, posp, Wn1a, Wn1b, bn1, Wn2, bn2, nW1a, nW1b, nb1):
    N, H = h.shape
    BN = 2000
    grid = (N // BN,)
    has_next = nW1a is not None
    np_ = len(parts)

    def body(*refs):
        h_r = refs[0]
        part_rs = refs[1:1 + np_]
        (posp_r, Wn1a_r, Wn1b_r, bn1_r, Wn2_r, bn2_r) = refs[1 + np_:7 + np_]
        k = 7 + np_
        if has_next:
            nW1a_r, nW1b_r, nb1_r = refs[k:k + 3]
            k += 3
        out = refs[k:]
        acc = part_rs[0][...]
        for pr in part_rs[1:]:
            acc = acc + pr[...]
        accM = acc[:, :H]
        accR = acc[:, H:]
        lane = lax.broadcasted_iota(jnp.int32, (1, 16), 1)
        cnt = jnp.sum(jnp.where(lane == 3, accR, 0.0), axis=1, keepdims=True)
        colmask = (lane < 3).astype(F32)
        upd = accR * colmask / jnp.maximum(cnt, 1.0)
        pp = posp_r[...] + upd
        t = _silu(_dot(h_r[...], Wn1a_r[...], preferred_element_type=F32)
                  + _dot(accM, Wn1b_r[...], preferred_element_type=F32)
                  + bn1_r[...])
        hn = _dot(t, Wn2_r[...], preferred_element_type=F32) + bn2_r[...]
        out[0][...] = hn
        out[1][...] = pp
        if has_next:
            A = _dot(hn, nW1a_r[...], preferred_element_type=F32) + nb1_r[...]
            B = _dot(hn, nW1b_r[...], preferred_element_type=F32)
            out[2][...] = jnp.concatenate([A, pp], axis=1)
            out[3][...] = jnp.concatenate([B, pp], axis=1)

    full = lambda a, b: pl.BlockSpec((a, b), lambda i: (0, 0))
    row = lambda w: pl.BlockSpec((BN, w), lambda i: (i, 0))
    in_specs = [row(H)] + [row(_TW)] * np_ + [row(16),
                full(H, H), full(H, H), full(1, H), full(H, H), full(1, H)]
    ins = [h] + list(parts) + [posp, Wn1a, Wn1b, bn1, Wn2, bn2]
    out_specs = [row(H), row(16)]
    out_shape = [jax.ShapeDtypeStruct((N, H), F32),
                 jax.ShapeDtypeStruct((N, 16), F32)]
    if has_next:
        in_specs += [full(H, H), full(H, H), full(1, H)]
        ins += [nW1a, nW1b, nb1]
        out_specs += [row(_TW), row(_TW)]
        out_shape += [jax.ShapeDtypeStruct((N, _TW), F32),
                      jax.ShapeDtypeStruct((N, _TW), F32)]
    return pl.pallas_call(
        body, grid=grid, in_specs=in_specs, out_specs=out_specs,
        out_shape=out_shape,
    )(*ins)


def _sc_gather(TA, TB, src2, dst2):
    N = TA.shape[0]
    nrow, C = src2.shape
    E = nrow * C
    epw = E // _NW
    nch = epw // C
    NB = 4
    nq = nch // NB
    rem = nch - nq * NB

    mesh = plsc.VectorSubcoreMesh(**_MESH)

    buf_set = [
        pltpu.VMEM((C, _TW), F32),
        pltpu.VMEM((C, _TW), F32),
        pltpu.SemaphoreType.DMA,
        pltpu.SemaphoreType.DMA,
    ]

    @functools.partial(
        pl.kernel,
        out_type=(
            jax.ShapeDtypeStruct((E, _TW), F32),
            jax.ShapeDtypeStruct((E, _TW), F32),
        ),
        mesh=mesh,
        compiler_params=_SC_PARAMS,
        scratch_types=buf_set * NB + [
            pltpu.VMEM((nch, C), jnp.int32),
            pltpu.VMEM((nch, C), jnp.int32),
        ],
    )
    def k(TA_h, TB_h, src_h, dst_h, oA, oB, *scr):
        sets = tuple(scr[4 * i:4 * i + 4] for i in range(NB))
        idxd, idxs = scr[4 * NB], scr[4 * NB + 1]
        wid = lax.axis_index("s") * _NC + lax.axis_index("c")
        base0 = wid * epw
        row0 = wid * nch
        pltpu.sync_copy(dst_h.at[pl.ds(row0, nch)], idxd)
        pltpu.sync_copy(src_h.at[pl.ds(row0, nch)], idxs)

        def issue(c, bset):
            bA, bB, gsem, _ = bset
            pltpu.async_copy(TA_h.at[idxd.at[c]], bA, gsem)
            pltpu.async_copy(TB_h.at[idxs.at[c]], bB, gsem)

        def gwait(c, bset):
            bA, bB, gsem, _ = bset
            pltpu.make_async_copy(TA_h.at[idxd.at[c]], bA, gsem).wait()
            pltpu.make_async_copy(TB_h.at[idxs.at[c]], bB, gsem).wait()

        def wdrain(bset):
            bA, bB, _, wsem = bset
            pltpu.make_async_copy(bA, oA.at[pl.ds(base0, C)], wsem).wait()
            pltpu.make_async_copy(bB, oB.at[pl.ds(base0, C)], wsem).wait()

        def writeback(c, bset):
            bA, bB, _, wsem = bset
            base = base0 + c * C
            pltpu.async_copy(bA, oA.at[pl.ds(base, C)], wsem)
            pltpu.async_copy(bB, oB.at[pl.ds(base, C)], wsem)

        def step(c, b):
            gwait(c, sets[b])
            writeback(c, sets[b])

            @pl.when(c >= 2)
            def _():
                wdrain(sets[(b + 2) % NB])

            @pl.when(c + 2 < nch)
            def _():
                issue(c + 2, sets[(b + 2) % NB])

        issue(0, sets[0])
        issue(1, sets[1])

        def qstep(kk, carry):
            for b in range(NB):
                step(NB * kk + b, b)
            return carry

        lax.fori_loop(0, nq, qstep, 0)
        for b in range(rem):
            step(nq * NB + b, b)
        wdrain(sets[(nch - 2) % NB])
        wdrain(sets[(nch - 1) % NB])

    return k(TA, TB, src2, dst2)


def _sc_scatter(msg, dst2, zP):
    N = zP.shape[0]
    nrow, C = dst2.shape
    E = nrow * C
    epw = E // _NW
    nch = epw // C
    nq = nch // 2
    rem = nch - nq * 2
    rpw = N // _NS

    mesh = plsc.VectorSubcoreMesh(**_MESH)

    buf_set = [
        pltpu.VMEM((C, _TW), F32),
        pltpu.SemaphoreType.DMA,
    ]

    @functools.partial(
        pl.kernel,
        out_type=jax.ShapeDtypeStruct((_NC * N, _TW), F32),
        mesh=mesh,
        compiler_params=_SC_PARAMS,
        scratch_types=buf_set + buf_set + [
            pltpu.VMEM((nch, C), jnp.int32),
            pltpu.VMEM_SHARED((N, _TW), F32),
        ],
    )
    def k(msg_h, dst_h, zP_h, oP, bM0, sem0, bM1, sem1, idx, acc):
        sets = ((bM0, sem0), (bM1, sem1))
        cid = lax.axis_index("c")
        sid = lax.axis_index("s")
        wid = sid * _NC + cid
        base0 = wid * epw
        row0 = wid * nch
        r0 = sid * rpw
        pltpu.sync_copy(zP_h.at[pl.ds(r0, rpw)], acc.at[pl.ds(r0, rpw)])
        pltpu.sync_copy(dst_h.at[pl.ds(row0, nch)], idx)
        plsc.subcore_barrier()

        def issue(c, bset):
            bM, sem = bset
            pltpu.async_copy(msg_h.at[pl.ds(base0 + c * C, C)], bM, sem)

        def consume(c, bset):
            bM, sem = bset
            pltpu.make_async_copy(msg_h.at[pl.ds(base0 + c * C, C)], bM, sem).wait()
            pltpu.sync_copy(bM, acc.at[idx.at[c]], add=True)

        issue(0, sets[0])
        issue(1, sets[1])

        def dstep(kk, carry):
            for b in range(2):
                c = 2 * kk + b
                consume(c, sets[b])

                @pl.when(c + 2 < nch)
                def _():
                    issue(c + 2, sets[b])

            return carry

        lax.fori_loop(0, nq, dstep, 0)
        for b in range(rem):
            consume(nq * 2 + b, sets[b])
        plsc.subcore_barrier()
        pltpu.sync_copy(acc.at[pl.ds(r0, rpw)], oP.at[pl.ds(cid * N + r0, rpw)])

    return k(msg, dst2, zP)


def kernel(edge_index, x, pos, edge_attr, params):
    src = edge_index[0]
    dst = edge_index[1]
    N = x.shape[0]
    E = src.shape[0]
    H = params["emb"]["W"][0].shape[1]

    posp = jnp.pad(pos, ((0, 0), (0, 16 - pos.shape[1])))
    zP = jnp.zeros((N, _TW), F32)
    src2 = src.reshape(E // _CHUNK, _CHUNK)
    dst2 = dst.reshape(E // _CHUNK, _CHUNK)
    E1 = (E * 3 // 5) // (_NW * _CHUNK) * (_NW * _CHUNK)
    r1 = E1 // _CHUNK
    splits = ((src2[:r1], dst2[:r1], edge_attr[:E1]),
              (src2[r1:], dst2[r1:], edge_attr[E1:]))

    layers = params["layers"]
    emb = params["emb"]

    def edge_w(lp):
        W1 = lp["edge"]["W"][0]
        return (W1[:H], W1[H:2 * H], W1[2 * H:2 * H + 1],
                W1[2 * H + 1:], lp["edge"]["b"][0].reshape(1, H))

    W1a0, W1b0, _, _, b10 = edge_w(layers[0])
    h, TA, TB = _tc_embed(x, posp, emb["W"][0], emb["b"][0].reshape(1, H),
                          W1a0, W1b0, b10)

    for li, lp in enumerate(layers):
        _, _, w1c, W1d, _ = edge_w(lp)
        ew = (w1c, W1d,
              lp["edge"]["W"][1], lp["edge"]["b"][1].reshape(1, H),
              lp["pos"]["W"][0], lp["pos"]["b"][0].reshape(1, H),
              lp["pos"]["W"][1].reshape(1, H), lp["pos"]["b"][1].reshape(1, 1))
        g1 = _sc_gather(TA, TB, splits[0][0], splits[0][1])
        g2 = _sc_gather(TA, TB, splits[1][0], splits[1][1])
        msg1 = _tc_edge(g1[0], g1[1], splits[0][2], *ew)
        part1 = _sc_scatter(msg1, splits[0][1], zP)
        msg2 = _tc_edge(g2[0], g2[1], splits[1][2], *ew)
        part2 = _sc_scatter(msg2, splits[1][1], zP)
        Wn1 = lp["node"]["W"][0]
        is_last = li == len(layers) - 1
        if not is_last:
            nW1a, nW1b, _, _, nb1 = edge_w(layers[li + 1])
        else:
            nW1a = nW1b = nb1 = None
        outs = _tc_node(
            h, (part1[:N], part1[N:], part2[:N], part2[N:]), posp,
            Wn1[:H], Wn1[H:], lp["node"]["b"][0].reshape(1, H),
            lp["node"]["W"][1], lp["node"]["b"][1].reshape(1, H),
            nW1a, nW1b, nb1,
        )
        if not is_last:
            h, posp, TA, TB = outs
        else:
            h, posp = outs

    return h, posp[:, :3]
